# baseline (speedup 1.0000x reference)
import functools

import jax
import jax.numpy as jnp
from jax import lax
from jax.experimental import pallas as pl
from jax.experimental.pallas import tpu as pltpu
from jax.experimental.pallas import tpu_sc as plsc


_NC = 2
_NS = 16
_ZR = 256


@functools.lru_cache(maxsize=None)
def _make_sc_op(rows, cols, dtype_name):
    dtype = jnp.dtype(dtype_name)
    nw = _NC * _NS
    rpw = rows // nw
    nwin = rpw // _ZR
    mesh = plsc.VectorSubcoreMesh(
        core_axis_name="c", subcore_axis_name="s",
        num_cores=_NC, num_subcores=_NS,
    )

    @functools.partial(
        pl.kernel,
        out_type=jax.ShapeDtypeStruct((2 * rows, cols), dtype),
        mesh=mesh,
        scratch_types=[
            pltpu.VMEM((_ZR, cols), dtype),
            pltpu.VMEM((_ZR, cols), dtype),
            pltpu.VMEM((_ZR, cols), dtype),
            pltpu.SemaphoreType.DMA,
            pltpu.SemaphoreType.DMA,
            pltpu.SemaphoreType.DMA,
            pltpu.SemaphoreType.DMA,
            pltpu.SemaphoreType.DMA,
        ],
    )
    def sc_op(x_hbm, o_hbm, bufa, bufb, zbuf, sia, sib, soa, sob, zsem):
        wid = lax.axis_index("s") * _NC + lax.axis_index("c")
        base = wid * rpw
        zero16 = jnp.zeros((16,), dtype)

        def zrow(i, carry):
            for j in range(cols // 16):
                zbuf[i, pl.ds(16 * j, 16)] = zero16
            return carry

        lax.fori_loop(0, _ZR, zrow, 0)

        zcopies = [
            pltpu.make_async_copy(
                zbuf,
                o_hbm.at[pl.ds(rows + base + k * _ZR, _ZR), :],
                zsem,
            )
            for k in range(nwin)
        ]
        for zc in zcopies:
            zc.start()

        bufs = [bufa, bufb]
        isems = [sia, sib]
        osems = [soa, sob]
        ins = [
            pltpu.make_async_copy(
                x_hbm.at[pl.ds(base + k * _ZR, _ZR), :],
                bufs[k % 2],
                isems[k % 2],
            )
            for k in range(nwin)
        ]
        outs = [
            pltpu.make_async_copy(
                bufs[k % 2],
                o_hbm.at[pl.ds(base + k * _ZR, _ZR), :],
                osems[k % 2],
            )
            for k in range(nwin)
        ]

        ins[0].start()
        for k in range(nwin):
            ins[k].wait()
            outs[k].start()
            if k + 1 < nwin:
                if k >= 1:
                    outs[k - 1].wait()
                ins[k + 1].start()
        if nwin >= 2:
            outs[nwin - 2].wait()
        outs[nwin - 1].wait()

        for zc in zcopies:
            zc.wait()

    return sc_op


def kernel(psi):
    rows, cols = psi.shape
    return _make_sc_op(rows, cols, psi.dtype.name)(psi)
